# baseline (speedup 1.0000x reference)
import functools

import jax
import jax.numpy as jnp
from jax import lax
from jax.experimental import pallas as pl
from jax.experimental.pallas import tpu as pltpu
from jax.experimental.pallas import tpu_sc as plsc

K = 20
TILE = 256
F32 = jnp.float32


def _dot(a, b, dims):
    return lax.dot_general(a, b, (dims, ((), ())), preferred_element_type=F32)


def _p0_kernel(x_ref, w1a_ref, wz_ref, y_ref, z_ref):
    xb = x_ref[0]
    y_ref[0] = _dot(xb, w1a_ref[...], ((0,), (1,)))
    z_ref[0] = _dot(xb, wz_ref[...], ((0,), (1,)))


def _p1_kernel(xt_ref, xb_ref, idx_ref):
    b = pl.program_id(0)
    n_total = xb_ref.shape[2]
    xt = xt_ref[0]
    xb = xb_ref[0]

    inner = _dot(xt, xb, ((0,), (0,)))
    xxp = jnp.sum(xt * xt, axis=0)
    xxb = jnp.sum(xb * xb, axis=0)
    d = 2.0 * inner - xxp[:, None] - xxb[None, :]

    iota = lax.broadcasted_iota(jnp.int32, d.shape, 1)
    neg = jnp.float32(-jnp.inf)
    for j in range(K):
        vmax = jnp.max(d, axis=1, keepdims=True)
        cand = jnp.where(d == vmax, iota, n_total)
        m = jnp.min(cand, axis=1, keepdims=True)
        d = jnp.where(iota == m, neg, d)
        idx_ref[0, :, pl.ds(j, 1)] = m + b * n_total


def _sc_gather_kernel(e_per_w, chunk, y_hbm, idx_hbm, g_hbm,
                      idx_v, rows_v, sem):
    nc = 2
    wid = lax.axis_index("s") * nc + lax.axis_index("c")
    base = wid * e_per_w
    pltpu.sync_copy(idx_hbm.at[pl.ds(base, e_per_w)], idx_v)
    for i in range(e_per_w // chunk):
        off = i * chunk
        pltpu.async_copy(
            y_hbm.at[idx_v.at[pl.ds(off, chunk)]], rows_v, sem).wait()
        pltpu.sync_copy(rows_v, g_hbm.at[pl.ds(base + off, chunk)])


def _p15_kernel(g_ref, z_ref, sumh_ref, sumsq_ref):
    b = pl.program_id(0)
    t = pl.program_id(1)
    z = z_ref[0]
    sh = jnp.zeros((z.shape[1],), F32)
    sq = jnp.zeros((z.shape[1],), F32)
    for j in range(K):
        hj = g_ref[0, :, j, :] + z
        sh = sh + jnp.sum(hj, axis=0)
        sq = sq + jnp.sum(hj * hj, axis=0)

    @pl.when(jnp.logical_and(b == 0, t == 0))
    def _init():
        sumh_ref[...] = jnp.zeros_like(sumh_ref)
        sumsq_ref[...] = jnp.zeros_like(sumsq_ref)
    sumh_ref[0, :] = sumh_ref[0, :] + sh
    sumsq_ref[0, :] = sumsq_ref[0, :] + sq


def _p2_kernel(g_ref, z_ref, s1_ref, t1_ref,
               x1_ref, muhr_ref, mhr_ref):
    b = pl.program_id(0)
    t = pl.program_id(1)
    z = z_ref[0]
    s1 = s1_ref[...]
    t1 = t1_ref[...]
    hrs = []
    for j in range(K):
        h = g_ref[0, :, j, :] + z
        hrs.append(jnp.maximum(h * s1 + t1, 0.0))
    mx = hrs[0]
    for j in range(1, K):
        mx = jnp.maximum(mx, hrs[j])
    ssum = jnp.zeros_like(mx)
    num = jnp.zeros_like(mx)
    for j in range(K):
        e = jnp.exp(hrs[j] - mx)
        ssum = ssum + e
        num = num + hrs[j] * e
    x1_ref[0] = num / ssum

    mu = jnp.zeros((z.shape[1],), F32)
    mm = jnp.zeros((z.shape[1], z.shape[1]), F32)
    for j in range(K):
        mu = mu + jnp.sum(hrs[j], axis=0)
        mm = mm + _dot(hrs[j], hrs[j], ((0,), (0,)))

    @pl.when(jnp.logical_and(b == 0, t == 0))
    def _init():
        muhr_ref[...] = jnp.zeros_like(muhr_ref)
        mhr_ref[...] = jnp.zeros_like(mhr_ref)
    muhr_ref[0, :] = muhr_ref[0, :] + mu
    mhr_ref[...] = mhr_ref[...] + mm


def _p2b_kernel(mu_ref, mm_ref, w2_ref, g2_ref, b2_ref, n_samples,
                s2_ref, t2_ref):
    minv = jnp.float32(1.0 / n_samples)
    w2 = w2_ref[...]
    mean_hr = mu_ref[...] * minv
    mean_c = _dot(w2, mean_hr, ((1,), (1,)))
    u = _dot(w2, mm_ref[...] * minv, ((1,), (0,)))
    e2 = jnp.sum(u * w2, axis=1, keepdims=True)
    var = e2 - mean_c * mean_c
    rstd = lax.rsqrt(var + 1e-5)
    g2 = g2_ref[...]
    sc = g2 * rstd
    tc = b2_ref[...] - mean_c * sc
    s2_ref[...] = jnp.reshape(sc, s2_ref.shape)
    t2_ref[...] = jnp.reshape(tc, t2_ref.shape)


def _p3_kernel(g_ref, z_ref, s1_ref, t1_ref, s2_ref, t2_ref, w2_ref, x1_ref,
               x2_ref, muv_ref, mv_ref):
    b = pl.program_id(0)
    t = pl.program_id(1)
    z = z_ref[0]
    s1 = s1_ref[...]
    t1 = t1_ref[...]
    s2 = s2_ref[...]
    t2 = t2_ref[...]
    w2 = w2_ref[...]
    hr2s = []
    for j in range(K):
        h = g_ref[0, :, j, :] + z
        hr = jnp.maximum(h * s1 + t1, 0.0)
        h2 = _dot(hr, w2, ((1,), (1,)))
        hr2s.append(jnp.maximum(h2 * s2 + t2, 0.0))
    mx = hr2s[0]
    for j in range(1, K):
        mx = jnp.maximum(mx, hr2s[j])
    ssum = jnp.zeros_like(mx)
    num = jnp.zeros_like(mx)
    for j in range(K):
        e = jnp.exp(hr2s[j] - mx)
        ssum = ssum + e
        num = num + hr2s[j] * e
    x2 = num / ssum
    x2_ref[0] = x2

    v = jnp.concatenate([x1_ref[0], x2], axis=1)
    mu = jnp.sum(v, axis=0)
    mm = _dot(v, v, ((0,), (0,)))

    @pl.when(jnp.logical_and(b == 0, t == 0))
    def _init():
        muv_ref[...] = jnp.zeros_like(muv_ref)
        mv_ref[...] = jnp.zeros_like(mv_ref)
    muv_ref[0, :] = muv_ref[0, :] + mu
    mv_ref[...] = mv_ref[...] + mm


def _p3b_kernel(mu_ref, mm_ref, wo_ref, go_ref, bo_ref, n_samples,
                so_ref, to_ref):
    minv = jnp.float32(1.0 / n_samples)
    wo = wo_ref[...]
    mean_v = mu_ref[...] * minv
    mean_c = _dot(wo, mean_v, ((1,), (1,)))
    u = _dot(wo, mm_ref[...] * minv, ((1,), (0,)))
    e2 = jnp.sum(u * wo, axis=1, keepdims=True)
    var = e2 - mean_c * mean_c
    rstd = lax.rsqrt(var + 1e-5)
    go = go_ref[...]
    so_ref[...] = go * rstd
    to_ref[...] = bo_ref[...] - mean_c * go * rstd


def _p4_kernel(x1_ref, x2_ref, wo_ref, so_ref, to_ref, out_ref):
    v = jnp.concatenate([x1_ref[0], x2_ref[0]], axis=1)
    o = _dot(wo_ref[...], v, ((1,), (1,)))
    out_ref[0] = jnp.maximum(o * so_ref[...] + to_ref[...], 0.0)


def kernel(x, W1, g1, b1, W2, g2, b2, Wout, gout, bout):
    B, C, N = x.shape
    C1 = W1.shape[0]
    C2 = W2.shape[0]
    CO = Wout.shape[0]
    CV = Wout.shape[1]
    T = N // TILE
    W1a = W1[:, :C]
    Wz = W1[:, C:] - W1[:, :C]

    y, z = pl.pallas_call(
        _p0_kernel,
        grid=(B,),
        in_specs=[
            pl.BlockSpec((1, C, N), lambda b: (b, 0, 0)),
            pl.BlockSpec((C1, C), lambda b: (0, 0)),
            pl.BlockSpec((C1, C), lambda b: (0, 0)),
        ],
        out_specs=[
            pl.BlockSpec((1, N, C1), lambda b: (b, 0, 0)),
            pl.BlockSpec((1, N, C1), lambda b: (b, 0, 0)),
        ],
        out_shape=[
            jax.ShapeDtypeStruct((B, N, C1), F32),
            jax.ShapeDtypeStruct((B, N, C1), F32),
        ],
    )(x, W1a, Wz)

    idx = pl.pallas_call(
        _p1_kernel,
        grid=(B, T),
        in_specs=[
            pl.BlockSpec((1, C, TILE), lambda b, t: (b, 0, t)),
            pl.BlockSpec((1, C, N), lambda b, t: (b, 0, 0)),
        ],
        out_specs=pl.BlockSpec((1, TILE, K), lambda b, t: (b, t, 0)),
        out_shape=jax.ShapeDtypeStruct((B, N, K), jnp.int32),
    )(x, x)

    n_edges = B * N * K
    n_workers = 32
    e_per_w = n_edges // n_workers
    chunk = 1280
    mesh = plsc.VectorSubcoreMesh(core_axis_name="c", subcore_axis_name="s")
    gather = pl.kernel(
        functools.partial(_sc_gather_kernel, e_per_w, chunk),
        mesh=mesh,
        out_type=jax.ShapeDtypeStruct((n_edges, C1), F32),
        scratch_types=[
            pltpu.VMEM((e_per_w,), jnp.int32),
            pltpu.VMEM((chunk, C1), F32),
            pltpu.SemaphoreType.DMA,
        ],
        compiler_params=pltpu.CompilerParams(use_tc_tiling_on_sc=False),
    )
    g = gather(y.reshape(B * N, C1), idx.reshape(n_edges))
    g = g.reshape(B, N, K, C1)

    sumh, sumsq = pl.pallas_call(
        _p15_kernel,
        grid=(B, T),
        in_specs=[
            pl.BlockSpec((1, TILE, K, C1), lambda b, t: (b, t, 0, 0)),
            pl.BlockSpec((1, TILE, C1), lambda b, t: (b, t, 0)),
        ],
        out_specs=[
            pl.BlockSpec((1, C1), lambda b, t: (0, 0)),
            pl.BlockSpec((1, C1), lambda b, t: (0, 0)),
        ],
        out_shape=[
            jax.ShapeDtypeStruct((1, C1), F32),
            jax.ShapeDtypeStruct((1, C1), F32),
        ],
    )(g, z)

    m_edges = B * N * K
    mean1 = sumh / m_edges
    var1 = sumsq / m_edges - mean1 * mean1
    rstd1 = 1.0 / jnp.sqrt(var1 + 1e-5)
    s1 = g1.reshape(1, C1) * rstd1
    t1 = b1.reshape(1, C1) - mean1 * s1

    x1, muhr, mhr = pl.pallas_call(
        _p2_kernel,
        grid=(B, T),
        in_specs=[
            pl.BlockSpec((1, TILE, K, C1), lambda b, t: (b, t, 0, 0)),
            pl.BlockSpec((1, TILE, C1), lambda b, t: (b, t, 0)),
            pl.BlockSpec((1, C1), lambda b, t: (0, 0)),
            pl.BlockSpec((1, C1), lambda b, t: (0, 0)),
        ],
        out_specs=[
            pl.BlockSpec((1, TILE, C1), lambda b, t: (b, t, 0)),
            pl.BlockSpec((1, C1), lambda b, t: (0, 0)),
            pl.BlockSpec((C1, C1), lambda b, t: (0, 0)),
        ],
        out_shape=[
            jax.ShapeDtypeStruct((B, N, C1), F32),
            jax.ShapeDtypeStruct((1, C1), F32),
            jax.ShapeDtypeStruct((C1, C1), F32),
        ],
    )(g, z, s1, t1)

    s2, t2 = pl.pallas_call(
        lambda mu, mm, w2, g2r, b2r, s2o, t2o: _p2b_kernel(
            mu, mm, w2, g2r, b2r, m_edges, s2o, t2o),
        out_shape=[
            jax.ShapeDtypeStruct((1, C2), F32),
            jax.ShapeDtypeStruct((1, C2), F32),
        ],
    )(muhr, mhr, W2, g2.reshape(C2, 1), b2.reshape(C2, 1))

    x2, muv, mv = pl.pallas_call(
        _p3_kernel,
        grid=(B, T),
        in_specs=[
            pl.BlockSpec((1, TILE, K, C1), lambda b, t: (b, t, 0, 0)),
            pl.BlockSpec((1, TILE, C1), lambda b, t: (b, t, 0)),
            pl.BlockSpec((1, C1), lambda b, t: (0, 0)),
            pl.BlockSpec((1, C1), lambda b, t: (0, 0)),
            pl.BlockSpec((1, C2), lambda b, t: (0, 0)),
            pl.BlockSpec((1, C2), lambda b, t: (0, 0)),
            pl.BlockSpec((C2, C1), lambda b, t: (0, 0)),
            pl.BlockSpec((1, TILE, C1), lambda b, t: (b, t, 0)),
        ],
        out_specs=[
            pl.BlockSpec((1, TILE, C2), lambda b, t: (b, t, 0)),
            pl.BlockSpec((1, CV), lambda b, t: (0, 0)),
            pl.BlockSpec((CV, CV), lambda b, t: (0, 0)),
        ],
        out_shape=[
            jax.ShapeDtypeStruct((B, N, C2), F32),
            jax.ShapeDtypeStruct((1, CV), F32),
            jax.ShapeDtypeStruct((CV, CV), F32),
        ],
    )(g, z, s1, t1, s2, t2, W2, x1)

    so, to = pl.pallas_call(
        lambda mu, mm, wo, gor, bor, soo, too: _p3b_kernel(
            mu, mm, wo, gor, bor, B * N, soo, too),
        out_shape=[
            jax.ShapeDtypeStruct((CO, 1), F32),
            jax.ShapeDtypeStruct((CO, 1), F32),
        ],
    )(muv, mv, Wout, gout.reshape(CO, 1), bout.reshape(CO, 1))

    out = pl.pallas_call(
        _p4_kernel,
        grid=(B, T),
        in_specs=[
            pl.BlockSpec((1, TILE, C1), lambda b, t: (b, t, 0)),
            pl.BlockSpec((1, TILE, C2), lambda b, t: (b, t, 0)),
            pl.BlockSpec((CO, CV), lambda b, t: (0, 0)),
            pl.BlockSpec((CO, 1), lambda b, t: (0, 0)),
            pl.BlockSpec((CO, 1), lambda b, t: (0, 0)),
        ],
        out_specs=pl.BlockSpec((1, CO, TILE), lambda b, t: (b, 0, t)),
        out_shape=jax.ShapeDtypeStruct((B, CO, N), F32),
    )(x1, x2, Wout, so, to)
    return out


# speedup vs baseline: 1.1934x; 1.1934x over previous
import functools

import jax
import jax.numpy as jnp
from jax import lax
from jax.experimental import pallas as pl
from jax.experimental.pallas import tpu as pltpu
from jax.experimental.pallas import tpu_sc as plsc

K = 20
TILE = 256
F32 = jnp.float32


def _dot(a, b, dims):
    return lax.dot_general(a, b, (dims, ((), ())), preferred_element_type=F32)


def _p0_kernel(x_ref, w1a_ref, wz_ref, y_ref, z_ref):
    xb = x_ref[0]
    y_ref[0] = _dot(xb, w1a_ref[...], ((0,), (1,)))
    z_ref[0] = _dot(xb, wz_ref[...], ((0,), (1,)))


def _p1_kernel(xt_ref, xb_ref, yb_ref, z_ref, idx_ref, sumh_ref, sumsq_ref):
    b = pl.program_id(0)
    t = pl.program_id(1)
    n_total = xb_ref.shape[2]
    xt = xt_ref[0]
    xb = xb_ref[0]
    yb = yb_ref[0]
    z = z_ref[0]

    inner = _dot(xt, xb, ((0,), (0,)))
    xxp = jnp.sum(xt * xt, axis=0)
    xxb = jnp.sum(xb * xb, axis=0)
    d = 2.0 * inner - xxp[:, None] - xxb[None, :]

    iota = lax.broadcasted_iota(jnp.int32, d.shape, 1)
    neg = jnp.float32(-jnp.inf)
    for j in range(K):
        vmax = jnp.max(d, axis=1, keepdims=True)
        cand = jnp.where(d == vmax, iota, n_total)
        m = jnp.min(cand, axis=1, keepdims=True)
        d = jnp.where(iota == m, neg, d)
        idx_ref[0, :, pl.ds(j, 1)] = m + b * n_total

    khot = (d == neg).astype(F32)
    s = _dot(khot, yb, ((1,), (0,)))
    s2 = _dot(khot, yb * yb, ((1,), (0,)))
    sh = jnp.sum(s, axis=0) + K * jnp.sum(z, axis=0)
    sq = (jnp.sum(s2, axis=0) + 2.0 * jnp.sum(s * z, axis=0)
          + K * jnp.sum(z * z, axis=0))

    @pl.when(jnp.logical_and(b == 0, t == 0))
    def _init():
        sumh_ref[...] = jnp.zeros_like(sumh_ref)
        sumsq_ref[...] = jnp.zeros_like(sumsq_ref)
    sumh_ref[0, :] = sumh_ref[0, :] + sh
    sumsq_ref[0, :] = sumsq_ref[0, :] + sq


def _sc_gather_kernel(e_per_w, chunk, y_hbm, idx_hbm, g_hbm,
                      idx_v, rows0, rows1, sem0, sem1):
    nc = 2
    wid = lax.axis_index("s") * nc + lax.axis_index("c")
    base = wid * e_per_w
    n_chunks = e_per_w // chunk
    rows = (rows0, rows1)
    sems = (sem0, sem1)
    pltpu.sync_copy(idx_hbm.at[pl.ds(base, e_per_w)], idx_v)
    cp0 = pltpu.async_copy(y_hbm.at[idx_v.at[pl.ds(0, chunk)]],
                           rows0, sem0)
    copies = [cp0]
    for i in range(n_chunks):
        if i + 1 < n_chunks:
            copies.append(pltpu.async_copy(
                y_hbm.at[idx_v.at[pl.ds((i + 1) * chunk, chunk)]],
                rows[(i + 1) % 2], sems[(i + 1) % 2]))
        copies[i].wait()
        pltpu.sync_copy(rows[i % 2], g_hbm.at[pl.ds(base + i * chunk, chunk)])


def _p2_kernel(g_ref, z_ref, s1_ref, t1_ref,
               x1_ref, muhr_ref, mhr_ref):
    b = pl.program_id(0)
    t = pl.program_id(1)
    z = z_ref[0]
    s1 = s1_ref[...]
    t1 = t1_ref[...]
    hrs = []
    for j in range(K):
        h = g_ref[0, :, j, :] + z
        hrs.append(jnp.maximum(h * s1 + t1, 0.0))
    mx = hrs[0]
    for j in range(1, K):
        mx = jnp.maximum(mx, hrs[j])
    ssum = jnp.zeros_like(mx)
    num = jnp.zeros_like(mx)
    for j in range(K):
        e = jnp.exp(hrs[j] - mx)
        ssum = ssum + e
        num = num + hrs[j] * e
    x1_ref[0] = num / ssum

    mu = jnp.zeros((z.shape[1],), F32)
    mm = jnp.zeros((z.shape[1], z.shape[1]), F32)
    for j in range(K):
        mu = mu + jnp.sum(hrs[j], axis=0)
        mm = mm + _dot(hrs[j], hrs[j], ((0,), (0,)))

    @pl.when(jnp.logical_and(b == 0, t == 0))
    def _init():
        muhr_ref[...] = jnp.zeros_like(muhr_ref)
        mhr_ref[...] = jnp.zeros_like(mhr_ref)
    muhr_ref[0, :] = muhr_ref[0, :] + mu
    mhr_ref[...] = mhr_ref[...] + mm


def _p2b_kernel(mu_ref, mm_ref, w2_ref, g2_ref, b2_ref, n_samples,
                s2_ref, t2_ref):
    minv = jnp.float32(1.0 / n_samples)
    w2 = w2_ref[...]
    mean_hr = mu_ref[...] * minv
    mean_c = _dot(w2, mean_hr, ((1,), (1,)))
    u = _dot(w2, mm_ref[...] * minv, ((1,), (0,)))
    e2 = jnp.sum(u * w2, axis=1, keepdims=True)
    var = e2 - mean_c * mean_c
    rstd = lax.rsqrt(var + 1e-5)
    g2 = g2_ref[...]
    sc = g2 * rstd
    tc = b2_ref[...] - mean_c * sc
    s2_ref[...] = jnp.reshape(sc, s2_ref.shape)
    t2_ref[...] = jnp.reshape(tc, t2_ref.shape)


def _p3_kernel(g_ref, z_ref, s1_ref, t1_ref, s2_ref, t2_ref, w2_ref, x1_ref,
               x2_ref, muv_ref, mv_ref):
    b = pl.program_id(0)
    t = pl.program_id(1)
    z = z_ref[0]
    s1 = s1_ref[...]
    t1 = t1_ref[...]
    s2 = s2_ref[...]
    t2 = t2_ref[...]
    w2 = w2_ref[...]
    hr2s = []
    for j in range(K):
        h = g_ref[0, :, j, :] + z
        hr = jnp.maximum(h * s1 + t1, 0.0)
        h2 = _dot(hr, w2, ((1,), (1,)))
        hr2s.append(jnp.maximum(h2 * s2 + t2, 0.0))
    mx = hr2s[0]
    for j in range(1, K):
        mx = jnp.maximum(mx, hr2s[j])
    ssum = jnp.zeros_like(mx)
    num = jnp.zeros_like(mx)
    for j in range(K):
        e = jnp.exp(hr2s[j] - mx)
        ssum = ssum + e
        num = num + hr2s[j] * e
    x2 = num / ssum
    x2_ref[0] = x2

    v = jnp.concatenate([x1_ref[0], x2], axis=1)
    mu = jnp.sum(v, axis=0)
    mm = _dot(v, v, ((0,), (0,)))

    @pl.when(jnp.logical_and(b == 0, t == 0))
    def _init():
        muv_ref[...] = jnp.zeros_like(muv_ref)
        mv_ref[...] = jnp.zeros_like(mv_ref)
    muv_ref[0, :] = muv_ref[0, :] + mu
    mv_ref[...] = mv_ref[...] + mm


def _p3b_kernel(mu_ref, mm_ref, wo_ref, go_ref, bo_ref, n_samples,
                so_ref, to_ref):
    minv = jnp.float32(1.0 / n_samples)
    wo = wo_ref[...]
    mean_v = mu_ref[...] * minv
    mean_c = _dot(wo, mean_v, ((1,), (1,)))
    u = _dot(wo, mm_ref[...] * minv, ((1,), (0,)))
    e2 = jnp.sum(u * wo, axis=1, keepdims=True)
    var = e2 - mean_c * mean_c
    rstd = lax.rsqrt(var + 1e-5)
    go = go_ref[...]
    so_ref[...] = go * rstd
    to_ref[...] = bo_ref[...] - mean_c * go * rstd


def _p4_kernel(x1_ref, x2_ref, wo_ref, so_ref, to_ref, out_ref):
    v = jnp.concatenate([x1_ref[0], x2_ref[0]], axis=1)
    o = _dot(wo_ref[...], v, ((1,), (1,)))
    out_ref[0] = jnp.maximum(o * so_ref[...] + to_ref[...], 0.0)


def kernel(x, W1, g1, b1, W2, g2, b2, Wout, gout, bout):
    B, C, N = x.shape
    C1 = W1.shape[0]
    C2 = W2.shape[0]
    CO = Wout.shape[0]
    CV = Wout.shape[1]
    T = N // TILE
    W1a = W1[:, :C]
    Wz = W1[:, C:] - W1[:, :C]

    y, z = pl.pallas_call(
        _p0_kernel,
        grid=(B,),
        in_specs=[
            pl.BlockSpec((1, C, N), lambda b: (b, 0, 0)),
            pl.BlockSpec((C1, C), lambda b: (0, 0)),
            pl.BlockSpec((C1, C), lambda b: (0, 0)),
        ],
        out_specs=[
            pl.BlockSpec((1, N, C1), lambda b: (b, 0, 0)),
            pl.BlockSpec((1, N, C1), lambda b: (b, 0, 0)),
        ],
        out_shape=[
            jax.ShapeDtypeStruct((B, N, C1), F32),
            jax.ShapeDtypeStruct((B, N, C1), F32),
        ],
    )(x, W1a, Wz)

    idx, sumh, sumsq = pl.pallas_call(
        _p1_kernel,
        grid=(B, T),
        in_specs=[
            pl.BlockSpec((1, C, TILE), lambda b, t: (b, 0, t)),
            pl.BlockSpec((1, C, N), lambda b, t: (b, 0, 0)),
            pl.BlockSpec((1, N, C1), lambda b, t: (b, 0, 0)),
            pl.BlockSpec((1, TILE, C1), lambda b, t: (b, t, 0)),
        ],
        out_specs=[
            pl.BlockSpec((1, TILE, K), lambda b, t: (b, t, 0)),
            pl.BlockSpec((1, C1), lambda b, t: (0, 0)),
            pl.BlockSpec((1, C1), lambda b, t: (0, 0)),
        ],
        out_shape=[
            jax.ShapeDtypeStruct((B, N, K), jnp.int32),
            jax.ShapeDtypeStruct((1, C1), F32),
            jax.ShapeDtypeStruct((1, C1), F32),
        ],
    )(x, x, y, z)

    n_edges = B * N * K
    n_workers = 32
    e_per_w = n_edges // n_workers
    chunk = 640
    mesh = plsc.VectorSubcoreMesh(core_axis_name="c", subcore_axis_name="s")
    gather = pl.kernel(
        functools.partial(_sc_gather_kernel, e_per_w, chunk),
        mesh=mesh,
        out_type=jax.ShapeDtypeStruct((n_edges, C1), F32),
        scratch_types=[
            pltpu.VMEM((e_per_w,), jnp.int32),
            pltpu.VMEM((chunk, C1), F32),
            pltpu.VMEM((chunk, C1), F32),
            pltpu.SemaphoreType.DMA,
            pltpu.SemaphoreType.DMA,
        ],
        compiler_params=pltpu.CompilerParams(use_tc_tiling_on_sc=False),
    )
    g = gather(y.reshape(B * N, C1), idx.reshape(n_edges))
    g = g.reshape(B, N, K, C1)

    m_edges = B * N * K
    mean1 = sumh / m_edges
    var1 = sumsq / m_edges - mean1 * mean1
    rstd1 = 1.0 / jnp.sqrt(var1 + 1e-5)
    s1 = g1.reshape(1, C1) * rstd1
    t1 = b1.reshape(1, C1) - mean1 * s1

    x1, muhr, mhr = pl.pallas_call(
        _p2_kernel,
        grid=(B, T),
        in_specs=[
            pl.BlockSpec((1, TILE, K, C1), lambda b, t: (b, t, 0, 0)),
            pl.BlockSpec((1, TILE, C1), lambda b, t: (b, t, 0)),
            pl.BlockSpec((1, C1), lambda b, t: (0, 0)),
            pl.BlockSpec((1, C1), lambda b, t: (0, 0)),
        ],
        out_specs=[
            pl.BlockSpec((1, TILE, C1), lambda b, t: (b, t, 0)),
            pl.BlockSpec((1, C1), lambda b, t: (0, 0)),
            pl.BlockSpec((C1, C1), lambda b, t: (0, 0)),
        ],
        out_shape=[
            jax.ShapeDtypeStruct((B, N, C1), F32),
            jax.ShapeDtypeStruct((1, C1), F32),
            jax.ShapeDtypeStruct((C1, C1), F32),
        ],
    )(g, z, s1, t1)

    s2, t2 = pl.pallas_call(
        lambda mu, mm, w2, g2r, b2r, s2o, t2o: _p2b_kernel(
            mu, mm, w2, g2r, b2r, m_edges, s2o, t2o),
        out_shape=[
            jax.ShapeDtypeStruct((1, C2), F32),
            jax.ShapeDtypeStruct((1, C2), F32),
        ],
    )(muhr, mhr, W2, g2.reshape(C2, 1), b2.reshape(C2, 1))

    x2, muv, mv = pl.pallas_call(
        _p3_kernel,
        grid=(B, T),
        in_specs=[
            pl.BlockSpec((1, TILE, K, C1), lambda b, t: (b, t, 0, 0)),
            pl.BlockSpec((1, TILE, C1), lambda b, t: (b, t, 0)),
            pl.BlockSpec((1, C1), lambda b, t: (0, 0)),
            pl.BlockSpec((1, C1), lambda b, t: (0, 0)),
            pl.BlockSpec((1, C2), lambda b, t: (0, 0)),
            pl.BlockSpec((1, C2), lambda b, t: (0, 0)),
            pl.BlockSpec((C2, C1), lambda b, t: (0, 0)),
            pl.BlockSpec((1, TILE, C1), lambda b, t: (b, t, 0)),
        ],
        out_specs=[
            pl.BlockSpec((1, TILE, C2), lambda b, t: (b, t, 0)),
            pl.BlockSpec((1, CV), lambda b, t: (0, 0)),
            pl.BlockSpec((CV, CV), lambda b, t: (0, 0)),
        ],
        out_shape=[
            jax.ShapeDtypeStruct((B, N, C2), F32),
            jax.ShapeDtypeStruct((1, CV), F32),
            jax.ShapeDtypeStruct((CV, CV), F32),
        ],
    )(g, z, s1, t1, s2, t2, W2, x1)

    so, to = pl.pallas_call(
        lambda mu, mm, wo, gor, bor, soo, too: _p3b_kernel(
            mu, mm, wo, gor, bor, B * N, soo, too),
        out_shape=[
            jax.ShapeDtypeStruct((CO, 1), F32),
            jax.ShapeDtypeStruct((CO, 1), F32),
        ],
    )(muv, mv, Wout, gout.reshape(CO, 1), bout.reshape(CO, 1))

    out = pl.pallas_call(
        _p4_kernel,
        grid=(B, T),
        in_specs=[
            pl.BlockSpec((1, TILE, C1), lambda b, t: (b, t, 0)),
            pl.BlockSpec((1, TILE, C2), lambda b, t: (b, t, 0)),
            pl.BlockSpec((CO, CV), lambda b, t: (0, 0)),
            pl.BlockSpec((CO, 1), lambda b, t: (0, 0)),
            pl.BlockSpec((CO, 1), lambda b, t: (0, 0)),
        ],
        out_specs=pl.BlockSpec((1, CO, TILE), lambda b, t: (b, 0, t)),
        out_shape=jax.ShapeDtypeStruct((B, CO, N), F32),
    )(x1, x2, Wout, so, to)
    return out


# speedup vs baseline: 2.0131x; 1.6868x over previous
import functools

import jax
import jax.numpy as jnp
from jax import lax
from jax.experimental import pallas as pl
from jax.experimental.pallas import tpu as pltpu
from jax.experimental.pallas import tpu_sc as plsc

K = 20
TILE = 256
F32 = jnp.float32


def _dot(a, b, dims):
    return lax.dot_general(a, b, (dims, ((), ())), preferred_element_type=F32)


def _p0_kernel(x_ref, w1a_ref, wz_ref, y_ref, z_ref):
    xb = x_ref[0]
    y_ref[0] = _dot(xb, w1a_ref[...], ((0,), (1,)))
    z_ref[0] = _dot(xb, wz_ref[...], ((0,), (1,)))


def _p1_kernel(xt_ref, xb_ref, yb_ref, z_ref, idx_ref, sumh_ref, sumsq_ref):
    b = pl.program_id(0)
    t = pl.program_id(1)
    n_total = xb_ref.shape[2]
    xt = xt_ref[0]
    xb = xb_ref[0]
    yb = yb_ref[0]
    z = z_ref[0]

    inner = _dot(xt, xb, ((0,), (0,)))
    xxp = jnp.sum(xt * xt, axis=0)
    xxb = jnp.sum(xb * xb, axis=0)
    d = 2.0 * inner - xxp[:, None] - xxb[None, :]

    iota = lax.broadcasted_iota(jnp.int32, d.shape, 1)
    neg = jnp.float32(-jnp.inf)
    for j in range(K):
        vmax = jnp.max(d, axis=1, keepdims=True)
        cand = jnp.where(d == vmax, iota, n_total)
        m = jnp.min(cand, axis=1, keepdims=True)
        d = jnp.where(iota == m, neg, d)
        idx_ref[0, :, pl.ds(j, 1)] = m + b * n_total

    khot = (d == neg).astype(F32)
    s = _dot(khot, yb, ((1,), (0,)))
    s2 = _dot(khot, yb * yb, ((1,), (0,)))
    sh = jnp.sum(s, axis=0) + K * jnp.sum(z, axis=0)
    sq = (jnp.sum(s2, axis=0) + 2.0 * jnp.sum(s * z, axis=0)
          + K * jnp.sum(z * z, axis=0))

    @pl.when(jnp.logical_and(b == 0, t == 0))
    def _init():
        sumh_ref[...] = jnp.zeros_like(sumh_ref)
        sumsq_ref[...] = jnp.zeros_like(sumsq_ref)
    sumh_ref[0, :] = sumh_ref[0, :] + sh
    sumsq_ref[0, :] = sumsq_ref[0, :] + sq


def _sc_gather_kernel(e_per_w, chunk, y_hbm, idx_hbm, g_hbm,
                      idx_v, rows0, rows1, sem0, sem1):
    nc = 2
    wid = lax.axis_index("s") * nc + lax.axis_index("c")
    base = wid * e_per_w
    n_chunks = e_per_w // chunk
    rows = (rows0, rows1)
    sems = (sem0, sem1)
    pltpu.sync_copy(idx_hbm.at[pl.ds(base, e_per_w)], idx_v)
    cp0 = pltpu.async_copy(y_hbm.at[idx_v.at[pl.ds(0, chunk)]],
                           rows0, sem0)
    copies = [cp0]
    for i in range(n_chunks):
        if i + 1 < n_chunks:
            copies.append(pltpu.async_copy(
                y_hbm.at[idx_v.at[pl.ds((i + 1) * chunk, chunk)]],
                rows[(i + 1) % 2], sems[(i + 1) % 2]))
        copies[i].wait()
        pltpu.sync_copy(rows[i % 2], g_hbm.at[pl.ds(base + i * chunk, chunk)])


def _p2_kernel(g_ref, z_ref, s1_ref, t1_ref,
               x1_ref, muhr_ref, mhr_ref):
    b = pl.program_id(0)
    t = pl.program_id(1)
    c1 = z_ref.shape[2]
    z = z_ref[0]
    z2 = jnp.concatenate([z, z], axis=1)
    s1 = s1_ref[...]
    t1 = t1_ref[...]
    s12 = jnp.concatenate([s1, s1], axis=1)
    t12 = jnp.concatenate([t1, t1], axis=1)
    hps = []
    for j2 in range(K // 2):
        hp = jnp.maximum((g_ref[0, j2] + z2) * s12 + t12, 0.0)
        hps.append(hp)
    mp = hps[0]
    for j2 in range(1, K // 2):
        mp = jnp.maximum(mp, hps[j2])
    mx = jnp.maximum(mp[:, :c1], mp[:, c1:])
    mx2 = jnp.concatenate([mx, mx], axis=1)
    ssum = jnp.zeros_like(mp)
    num = jnp.zeros_like(mp)
    mu = jnp.zeros((2 * c1,), F32)
    mm = jnp.zeros((2 * c1, 2 * c1), F32)
    for j2 in range(K // 2):
        e = jnp.exp(hps[j2] - mx2)
        ssum = ssum + e
        num = num + hps[j2] * e
        mu = mu + jnp.sum(hps[j2], axis=0)
        mm = mm + _dot(hps[j2], hps[j2], ((0,), (0,)))
    x1_ref[0] = ((num[:, :c1] + num[:, c1:])
                 / (ssum[:, :c1] + ssum[:, c1:]))

    @pl.when(jnp.logical_and(b == 0, t == 0))
    def _init():
        muhr_ref[...] = jnp.zeros_like(muhr_ref)
        mhr_ref[...] = jnp.zeros_like(mhr_ref)
    muhr_ref[0, :] = muhr_ref[0, :] + mu
    mhr_ref[...] = mhr_ref[...] + mm


def _p2b_kernel(mu_ref, mm_ref, w2_ref, g2_ref, b2_ref, n_samples,
                s2_ref, t2_ref):
    minv = jnp.float32(1.0 / n_samples)
    c1 = w2_ref.shape[1]
    w2 = w2_ref[...]
    mup = mu_ref[...]
    mmp = mm_ref[...]
    mean_hr = (mup[:, :c1] + mup[:, c1:]) * minv
    mm_hr = (mmp[:c1, :c1] + mmp[c1:, c1:]) * minv
    mean_c = _dot(w2, mean_hr, ((1,), (1,)))
    u = _dot(w2, mm_hr, ((1,), (0,)))
    e2 = jnp.sum(u * w2, axis=1, keepdims=True)
    var = e2 - mean_c * mean_c
    rstd = lax.rsqrt(var + 1e-5)
    gg = g2_ref[...]
    sc = gg * rstd
    tc = b2_ref[...] - mean_c * sc
    s2_ref[...] = jnp.reshape(sc, s2_ref.shape)
    t2_ref[...] = jnp.reshape(tc, t2_ref.shape)


def _p3_kernel(g_ref, z_ref, s1_ref, t1_ref, s2_ref, t2_ref, w2p_ref, x1_ref,
               x2_ref, muv_ref, mv_ref):
    b = pl.program_id(0)
    t = pl.program_id(1)
    c1 = z_ref.shape[2]
    c2 = s2_ref.shape[1]
    z = z_ref[0]
    z2 = jnp.concatenate([z, z], axis=1)
    s1 = s1_ref[...]
    t1 = t1_ref[...]
    s12 = jnp.concatenate([s1, s1], axis=1)
    t12 = jnp.concatenate([t1, t1], axis=1)
    s2 = s2_ref[...]
    t2 = t2_ref[...]
    s22 = jnp.concatenate([s2, s2], axis=1)
    t22 = jnp.concatenate([t2, t2], axis=1)
    w2p = w2p_ref[...]
    hr2s = []
    for j2 in range(K // 2):
        hp = jnp.maximum((g_ref[0, j2] + z2) * s12 + t12, 0.0)
        h2p = _dot(hp, w2p, ((1,), (0,)))
        hr2s.append(jnp.maximum(h2p * s22 + t22, 0.0))
    mp = hr2s[0]
    for j2 in range(1, K // 2):
        mp = jnp.maximum(mp, hr2s[j2])
    mx = jnp.maximum(mp[:, :c2], mp[:, c2:])
    mx2 = jnp.concatenate([mx, mx], axis=1)
    ssum = jnp.zeros_like(mp)
    num = jnp.zeros_like(mp)
    for j2 in range(K // 2):
        e = jnp.exp(hr2s[j2] - mx2)
        ssum = ssum + e
        num = num + hr2s[j2] * e
    x2 = ((num[:, :c2] + num[:, c2:])
          / (ssum[:, :c2] + ssum[:, c2:]))
    x2_ref[0] = x2

    v = jnp.concatenate([x1_ref[0], x2], axis=1)
    mu = jnp.sum(v, axis=0)
    mm = _dot(v, v, ((0,), (0,)))

    @pl.when(jnp.logical_and(b == 0, t == 0))
    def _init():
        muv_ref[...] = jnp.zeros_like(muv_ref)
        mv_ref[...] = jnp.zeros_like(mv_ref)
    muv_ref[0, :] = muv_ref[0, :] + mu
    mv_ref[...] = mv_ref[...] + mm


def _p3b_kernel(mu_ref, mm_ref, wo_ref, go_ref, bo_ref, n_samples,
                so_ref, to_ref):
    minv = jnp.float32(1.0 / n_samples)
    wo = wo_ref[...]
    mean_v = mu_ref[...] * minv
    mean_c = _dot(wo, mean_v, ((1,), (1,)))
    u = _dot(wo, mm_ref[...] * minv, ((1,), (0,)))
    e2 = jnp.sum(u * wo, axis=1, keepdims=True)
    var = e2 - mean_c * mean_c
    rstd = lax.rsqrt(var + 1e-5)
    go = go_ref[...]
    so_ref[...] = go * rstd
    to_ref[...] = bo_ref[...] - mean_c * go * rstd


def _p4_kernel(x1_ref, x2_ref, wo_ref, so_ref, to_ref, out_ref):
    v = jnp.concatenate([x1_ref[0], x2_ref[0]], axis=1)
    o = _dot(wo_ref[...], v, ((1,), (1,)))
    out_ref[0] = jnp.maximum(o * so_ref[...] + to_ref[...], 0.0)


def kernel(x, W1, g1, b1, W2, g2, b2, Wout, gout, bout):
    B, C, N = x.shape
    C1 = W1.shape[0]
    C2 = W2.shape[0]
    CO = Wout.shape[0]
    CV = Wout.shape[1]
    T = N // TILE
    W1a = W1[:, :C]
    Wz = W1[:, C:] - W1[:, :C]

    y, z = pl.pallas_call(
        _p0_kernel,
        grid=(B,),
        in_specs=[
            pl.BlockSpec((1, C, N), lambda b: (b, 0, 0)),
            pl.BlockSpec((C1, C), lambda b: (0, 0)),
            pl.BlockSpec((C1, C), lambda b: (0, 0)),
        ],
        out_specs=[
            pl.BlockSpec((1, N, C1), lambda b: (b, 0, 0)),
            pl.BlockSpec((1, N, C1), lambda b: (b, 0, 0)),
        ],
        out_shape=[
            jax.ShapeDtypeStruct((B, N, C1), F32),
            jax.ShapeDtypeStruct((B, N, C1), F32),
        ],
    )(x, W1a, Wz)

    idx, sumh, sumsq = pl.pallas_call(
        _p1_kernel,
        grid=(B, T),
        in_specs=[
            pl.BlockSpec((1, C, TILE), lambda b, t: (b, 0, t)),
            pl.BlockSpec((1, C, N), lambda b, t: (b, 0, 0)),
            pl.BlockSpec((1, N, C1), lambda b, t: (b, 0, 0)),
            pl.BlockSpec((1, TILE, C1), lambda b, t: (b, t, 0)),
        ],
        out_specs=[
            pl.BlockSpec((1, TILE, K), lambda b, t: (b, t, 0)),
            pl.BlockSpec((1, C1), lambda b, t: (0, 0)),
            pl.BlockSpec((1, C1), lambda b, t: (0, 0)),
        ],
        out_shape=[
            jax.ShapeDtypeStruct((B, N, K), jnp.int32),
            jax.ShapeDtypeStruct((1, C1), F32),
            jax.ShapeDtypeStruct((1, C1), F32),
        ],
    )(x, x, y, z)

    n_edges = B * N * K
    n_workers = 32
    e_per_w = n_edges // n_workers
    chunk = 640
    mesh = plsc.VectorSubcoreMesh(core_axis_name="c", subcore_axis_name="s")
    gather = pl.kernel(
        functools.partial(_sc_gather_kernel, e_per_w, chunk),
        mesh=mesh,
        out_type=jax.ShapeDtypeStruct((n_edges, C1), F32),
        scratch_types=[
            pltpu.VMEM((e_per_w,), jnp.int32),
            pltpu.VMEM((chunk, C1), F32),
            pltpu.VMEM((chunk, C1), F32),
            pltpu.SemaphoreType.DMA,
            pltpu.SemaphoreType.DMA,
        ],
        compiler_params=pltpu.CompilerParams(use_tc_tiling_on_sc=False),
    )
    idxp = jnp.transpose(idx.reshape(B, N, K // 2, 2),
                         (0, 2, 1, 3)).reshape(n_edges)
    g = gather(y.reshape(B * N, C1), idxp)
    gpk = g.reshape(B, K // 2, N, 2 * C1)

    m_edges = B * N * K
    mean1 = sumh / m_edges
    var1 = sumsq / m_edges - mean1 * mean1
    rstd1 = 1.0 / jnp.sqrt(var1 + 1e-5)
    s1 = g1.reshape(1, C1) * rstd1
    t1 = b1.reshape(1, C1) - mean1 * s1

    x1, muhr, mhr = pl.pallas_call(
        _p2_kernel,
        grid=(B, T),
        in_specs=[
            pl.BlockSpec((1, K // 2, TILE, 2 * C1),
                         lambda b, t: (b, 0, t, 0)),
            pl.BlockSpec((1, TILE, C1), lambda b, t: (b, t, 0)),
            pl.BlockSpec((1, C1), lambda b, t: (0, 0)),
            pl.BlockSpec((1, C1), lambda b, t: (0, 0)),
        ],
        out_specs=[
            pl.BlockSpec((1, TILE, C1), lambda b, t: (b, t, 0)),
            pl.BlockSpec((1, 2 * C1), lambda b, t: (0, 0)),
            pl.BlockSpec((2 * C1, 2 * C1), lambda b, t: (0, 0)),
        ],
        out_shape=[
            jax.ShapeDtypeStruct((B, N, C1), F32),
            jax.ShapeDtypeStruct((1, 2 * C1), F32),
            jax.ShapeDtypeStruct((2 * C1, 2 * C1), F32),
        ],
    )(gpk, z, s1, t1)

    s2, t2 = pl.pallas_call(
        lambda mu, mm, w2, g2r, b2r, s2o, t2o: _p2b_kernel(
            mu, mm, w2, g2r, b2r, m_edges, s2o, t2o),
        out_shape=[
            jax.ShapeDtypeStruct((1, C2), F32),
            jax.ShapeDtypeStruct((1, C2), F32),
        ],
    )(muhr, mhr, W2, g2.reshape(C2, 1), b2.reshape(C2, 1))

    w2p = jnp.zeros((2 * C1, 2 * C2), F32)
    w2p = w2p.at[:C1, :C2].set(W2.T)
    w2p = w2p.at[C1:, C2:].set(W2.T)

    x2, muv, mv = pl.pallas_call(
        _p3_kernel,
        grid=(B, T),
        in_specs=[
            pl.BlockSpec((1, K // 2, TILE, 2 * C1),
                         lambda b, t: (b, 0, t, 0)),
            pl.BlockSpec((1, TILE, C1), lambda b, t: (b, t, 0)),
            pl.BlockSpec((1, C1), lambda b, t: (0, 0)),
            pl.BlockSpec((1, C1), lambda b, t: (0, 0)),
            pl.BlockSpec((1, C2), lambda b, t: (0, 0)),
            pl.BlockSpec((1, C2), lambda b, t: (0, 0)),
            pl.BlockSpec((2 * C1, 2 * C2), lambda b, t: (0, 0)),
            pl.BlockSpec((1, TILE, C1), lambda b, t: (b, t, 0)),
        ],
        out_specs=[
            pl.BlockSpec((1, TILE, C2), lambda b, t: (b, t, 0)),
            pl.BlockSpec((1, CV), lambda b, t: (0, 0)),
            pl.BlockSpec((CV, CV), lambda b, t: (0, 0)),
        ],
        out_shape=[
            jax.ShapeDtypeStruct((B, N, C2), F32),
            jax.ShapeDtypeStruct((1, CV), F32),
            jax.ShapeDtypeStruct((CV, CV), F32),
        ],
    )(gpk, z, s1, t1, s2, t2, w2p, x1)

    so, to = pl.pallas_call(
        lambda mu, mm, wo, gor, bor, soo, too: _p3b_kernel(
            mu, mm, wo, gor, bor, B * N, soo, too),
        out_shape=[
            jax.ShapeDtypeStruct((CO, 1), F32),
            jax.ShapeDtypeStruct((CO, 1), F32),
        ],
    )(muv, mv, Wout, gout.reshape(CO, 1), bout.reshape(CO, 1))

    out = pl.pallas_call(
        _p4_kernel,
        grid=(B, T),
        in_specs=[
            pl.BlockSpec((1, TILE, C1), lambda b, t: (b, t, 0)),
            pl.BlockSpec((1, TILE, C2), lambda b, t: (b, t, 0)),
            pl.BlockSpec((CO, CV), lambda b, t: (0, 0)),
            pl.BlockSpec((CO, 1), lambda b, t: (0, 0)),
            pl.BlockSpec((CO, 1), lambda b, t: (0, 0)),
        ],
        out_specs=pl.BlockSpec((1, CO, TILE), lambda b, t: (b, 0, t)),
        out_shape=jax.ShapeDtypeStruct((B, CO, N), F32),
    )(x1, x2, Wout, so, to)
    return out


# speedup vs baseline: 2.2631x; 1.1242x over previous
import functools

import jax
import jax.numpy as jnp
from jax import lax
from jax.experimental import pallas as pl
from jax.experimental.pallas import tpu as pltpu
from jax.experimental.pallas import tpu_sc as plsc

K = 20
TILE = 256
F32 = jnp.float32


def _dot(a, b, dims):
    return lax.dot_general(a, b, (dims, ((), ())), preferred_element_type=F32)


def _p0_kernel(x_ref, w1a_ref, wz_ref, y_ref, z_ref):
    xb = x_ref[0]
    y_ref[0] = _dot(xb, w1a_ref[...], ((0,), (1,)))
    z_ref[0] = _dot(xb, wz_ref[...], ((0,), (1,)))


def _p1_kernel(xt_ref, xb_ref, yb_ref, z_ref, idx_ref, sumh_ref, sumsq_ref):
    b = pl.program_id(0)
    t = pl.program_id(1)
    n_total = xb_ref.shape[2]
    xt = xt_ref[0]
    xb = xb_ref[0]
    yb = yb_ref[0]
    z = z_ref[0]

    inner = _dot(xt, xb, ((0,), (0,)))
    xxp = jnp.sum(xt * xt, axis=0)
    xxb = jnp.sum(xb * xb, axis=0)
    d = 2.0 * inner - xxp[:, None] - xxb[None, :]

    iota = lax.broadcasted_iota(jnp.int32, d.shape, 1)
    neg = jnp.float32(-jnp.inf)
    for j in range(K):
        m = jnp.argmax(d, axis=1).astype(jnp.int32)[:, None]
        d = jnp.where(iota == m, neg, d)
        idx_ref[0, :, pl.ds(j, 1)] = m + b * n_total

    khot = (d == neg).astype(F32)
    s = _dot(khot, yb, ((1,), (0,)))
    s2 = _dot(khot, yb * yb, ((1,), (0,)))
    sh = jnp.sum(s, axis=0) + K * jnp.sum(z, axis=0)
    sq = (jnp.sum(s2, axis=0) + 2.0 * jnp.sum(s * z, axis=0)
          + K * jnp.sum(z * z, axis=0))

    @pl.when(jnp.logical_and(b == 0, t == 0))
    def _init():
        sumh_ref[...] = jnp.zeros_like(sumh_ref)
        sumsq_ref[...] = jnp.zeros_like(sumsq_ref)
    sumh_ref[0, :] = sumh_ref[0, :] + sh
    sumsq_ref[0, :] = sumsq_ref[0, :] + sq


def _sc_gather_kernel(e_per_w, chunk, y_hbm, idx_hbm, g_hbm,
                      idx_v, rows0, rows1, sem0, sem1):
    nc = 2
    wid = lax.axis_index("s") * nc + lax.axis_index("c")
    base = wid * e_per_w
    n_chunks = e_per_w // chunk
    rows = (rows0, rows1)
    sems = (sem0, sem1)
    pltpu.sync_copy(idx_hbm.at[pl.ds(base, e_per_w)], idx_v)
    cp0 = pltpu.async_copy(y_hbm.at[idx_v.at[pl.ds(0, chunk)]],
                           rows0, sem0)
    copies = [cp0]
    for i in range(n_chunks):
        if i + 1 < n_chunks:
            copies.append(pltpu.async_copy(
                y_hbm.at[idx_v.at[pl.ds((i + 1) * chunk, chunk)]],
                rows[(i + 1) % 2], sems[(i + 1) % 2]))
        copies[i].wait()
        pltpu.sync_copy(rows[i % 2], g_hbm.at[pl.ds(base + i * chunk, chunk)])


def _p2_kernel(g_ref, z_ref, s1_ref, t1_ref,
               x1_ref, muhr_ref, mhr_ref):
    b = pl.program_id(0)
    t = pl.program_id(1)
    c1 = z_ref.shape[2]
    z = z_ref[0]
    z2 = jnp.concatenate([z, z], axis=1)
    s1 = s1_ref[...]
    t1 = t1_ref[...]
    s12 = jnp.concatenate([s1, s1], axis=1)
    t12 = jnp.concatenate([t1, t1], axis=1)
    hps = []
    for j2 in range(K // 2):
        hp = jnp.maximum((g_ref[0, j2] + z2) * s12 + t12, 0.0)
        hps.append(hp)
    mp = hps[0]
    for j2 in range(1, K // 2):
        mp = jnp.maximum(mp, hps[j2])
    mx = jnp.maximum(mp[:, :c1], mp[:, c1:])
    mx2 = jnp.concatenate([mx, mx], axis=1)
    ssum = jnp.zeros_like(mp)
    num = jnp.zeros_like(mp)
    mu = jnp.zeros((2 * c1,), F32)
    mm = jnp.zeros((2 * c1, 2 * c1), F32)
    for j2 in range(K // 2):
        e = jnp.exp(hps[j2] - mx2)
        ssum = ssum + e
        num = num + hps[j2] * e
        mu = mu + jnp.sum(hps[j2], axis=0)
        mm = mm + _dot(hps[j2], hps[j2], ((0,), (0,)))
    x1_ref[0] = ((num[:, :c1] + num[:, c1:])
                 / (ssum[:, :c1] + ssum[:, c1:]))

    @pl.when(jnp.logical_and(b == 0, t == 0))
    def _init():
        muhr_ref[...] = jnp.zeros_like(muhr_ref)
        mhr_ref[...] = jnp.zeros_like(mhr_ref)
    muhr_ref[0, :] = muhr_ref[0, :] + mu
    mhr_ref[...] = mhr_ref[...] + mm


def _p2b_kernel(mu_ref, mm_ref, w2_ref, g2_ref, b2_ref, n_samples,
                s2_ref, t2_ref):
    minv = jnp.float32(1.0 / n_samples)
    c1 = w2_ref.shape[1]
    w2 = w2_ref[...]
    mup = mu_ref[...]
    mmp = mm_ref[...]
    mean_hr = (mup[:, :c1] + mup[:, c1:]) * minv
    mm_hr = (mmp[:c1, :c1] + mmp[c1:, c1:]) * minv
    mean_c = _dot(w2, mean_hr, ((1,), (1,)))
    u = _dot(w2, mm_hr, ((1,), (0,)))
    e2 = jnp.sum(u * w2, axis=1, keepdims=True)
    var = e2 - mean_c * mean_c
    rstd = lax.rsqrt(var + 1e-5)
    gg = g2_ref[...]
    sc = gg * rstd
    tc = b2_ref[...] - mean_c * sc
    s2_ref[...] = jnp.reshape(sc, s2_ref.shape)
    t2_ref[...] = jnp.reshape(tc, t2_ref.shape)


def _p3_kernel(g_ref, z_ref, s1_ref, t1_ref, s2_ref, t2_ref, w2p_ref, x1_ref,
               x2_ref, muv_ref, mv_ref):
    b = pl.program_id(0)
    t = pl.program_id(1)
    c1 = z_ref.shape[2]
    c2 = s2_ref.shape[1]
    z = z_ref[0]
    z2 = jnp.concatenate([z, z], axis=1)
    s1 = s1_ref[...]
    t1 = t1_ref[...]
    s12 = jnp.concatenate([s1, s1], axis=1)
    t12 = jnp.concatenate([t1, t1], axis=1)
    s2 = s2_ref[...]
    t2 = t2_ref[...]
    s22 = jnp.concatenate([s2, s2], axis=1)
    t22 = jnp.concatenate([t2, t2], axis=1)
    w2p = w2p_ref[...]
    hr2s = []
    for j2 in range(K // 2):
        hp = jnp.maximum((g_ref[0, j2] + z2) * s12 + t12, 0.0)
        h2p = _dot(hp, w2p, ((1,), (0,)))
        hr2s.append(jnp.maximum(h2p * s22 + t22, 0.0))
    mp = hr2s[0]
    for j2 in range(1, K // 2):
        mp = jnp.maximum(mp, hr2s[j2])
    mx = jnp.maximum(mp[:, :c2], mp[:, c2:])
    mx2 = jnp.concatenate([mx, mx], axis=1)
    ssum = jnp.zeros_like(mp)
    num = jnp.zeros_like(mp)
    for j2 in range(K // 2):
        e = jnp.exp(hr2s[j2] - mx2)
        ssum = ssum + e
        num = num + hr2s[j2] * e
    x2 = ((num[:, :c2] + num[:, c2:])
          / (ssum[:, :c2] + ssum[:, c2:]))
    x2_ref[0] = x2

    v = jnp.concatenate([x1_ref[0], x2], axis=1)
    mu = jnp.sum(v, axis=0)
    mm = _dot(v, v, ((0,), (0,)))

    @pl.when(jnp.logical_and(b == 0, t == 0))
    def _init():
        muv_ref[...] = jnp.zeros_like(muv_ref)
        mv_ref[...] = jnp.zeros_like(mv_ref)
    muv_ref[0, :] = muv_ref[0, :] + mu
    mv_ref[...] = mv_ref[...] + mm


def _p3b_kernel(mu_ref, mm_ref, wo_ref, go_ref, bo_ref, n_samples,
                so_ref, to_ref):
    minv = jnp.float32(1.0 / n_samples)
    wo = wo_ref[...]
    mean_v = mu_ref[...] * minv
    mean_c = _dot(wo, mean_v, ((1,), (1,)))
    u = _dot(wo, mm_ref[...] * minv, ((1,), (0,)))
    e2 = jnp.sum(u * wo, axis=1, keepdims=True)
    var = e2 - mean_c * mean_c
    rstd = lax.rsqrt(var + 1e-5)
    go = go_ref[...]
    so_ref[...] = go * rstd
    to_ref[...] = bo_ref[...] - mean_c * go * rstd


def _p4_kernel(x1_ref, x2_ref, wo_ref, so_ref, to_ref, out_ref):
    v = jnp.concatenate([x1_ref[0], x2_ref[0]], axis=1)
    o = _dot(wo_ref[...], v, ((1,), (1,)))
    out_ref[0] = jnp.maximum(o * so_ref[...] + to_ref[...], 0.0)


def kernel(x, W1, g1, b1, W2, g2, b2, Wout, gout, bout):
    B, C, N = x.shape
    C1 = W1.shape[0]
    C2 = W2.shape[0]
    CO = Wout.shape[0]
    CV = Wout.shape[1]
    T = N // TILE
    W1a = W1[:, :C]
    Wz = W1[:, C:] - W1[:, :C]

    y, z = pl.pallas_call(
        _p0_kernel,
        grid=(B,),
        in_specs=[
            pl.BlockSpec((1, C, N), lambda b: (b, 0, 0)),
            pl.BlockSpec((C1, C), lambda b: (0, 0)),
            pl.BlockSpec((C1, C), lambda b: (0, 0)),
        ],
        out_specs=[
            pl.BlockSpec((1, N, C1), lambda b: (b, 0, 0)),
            pl.BlockSpec((1, N, C1), lambda b: (b, 0, 0)),
        ],
        out_shape=[
            jax.ShapeDtypeStruct((B, N, C1), F32),
            jax.ShapeDtypeStruct((B, N, C1), F32),
        ],
    )(x, W1a, Wz)

    idx, sumh, sumsq = pl.pallas_call(
        _p1_kernel,
        grid=(B, T),
        in_specs=[
            pl.BlockSpec((1, C, TILE), lambda b, t: (b, 0, t)),
            pl.BlockSpec((1, C, N), lambda b, t: (b, 0, 0)),
            pl.BlockSpec((1, N, C1), lambda b, t: (b, 0, 0)),
            pl.BlockSpec((1, TILE, C1), lambda b, t: (b, t, 0)),
        ],
        out_specs=[
            pl.BlockSpec((1, TILE, K), lambda b, t: (b, t, 0)),
            pl.BlockSpec((1, C1), lambda b, t: (0, 0)),
            pl.BlockSpec((1, C1), lambda b, t: (0, 0)),
        ],
        out_shape=[
            jax.ShapeDtypeStruct((B, N, K), jnp.int32),
            jax.ShapeDtypeStruct((1, C1), F32),
            jax.ShapeDtypeStruct((1, C1), F32),
        ],
    )(x, x, y, z)

    n_edges = B * N * K
    n_workers = 32
    e_per_w = n_edges // n_workers
    chunk = 640
    mesh = plsc.VectorSubcoreMesh(core_axis_name="c", subcore_axis_name="s")
    gather = pl.kernel(
        functools.partial(_sc_gather_kernel, e_per_w, chunk),
        mesh=mesh,
        out_type=jax.ShapeDtypeStruct((n_edges, C1), F32),
        scratch_types=[
            pltpu.VMEM((e_per_w,), jnp.int32),
            pltpu.VMEM((chunk, C1), F32),
            pltpu.VMEM((chunk, C1), F32),
            pltpu.SemaphoreType.DMA,
            pltpu.SemaphoreType.DMA,
        ],
        compiler_params=pltpu.CompilerParams(use_tc_tiling_on_sc=False),
    )
    idxp = jnp.transpose(idx.reshape(B, N, K // 2, 2),
                         (0, 2, 1, 3)).reshape(n_edges)
    g = gather(y.reshape(B * N, C1), idxp)
    gpk = g.reshape(B, K // 2, N, 2 * C1)

    m_edges = B * N * K
    mean1 = sumh / m_edges
    var1 = sumsq / m_edges - mean1 * mean1
    rstd1 = 1.0 / jnp.sqrt(var1 + 1e-5)
    s1 = g1.reshape(1, C1) * rstd1
    t1 = b1.reshape(1, C1) - mean1 * s1

    x1, muhr, mhr = pl.pallas_call(
        _p2_kernel,
        grid=(B, T),
        in_specs=[
            pl.BlockSpec((1, K // 2, TILE, 2 * C1),
                         lambda b, t: (b, 0, t, 0)),
            pl.BlockSpec((1, TILE, C1), lambda b, t: (b, t, 0)),
            pl.BlockSpec((1, C1), lambda b, t: (0, 0)),
            pl.BlockSpec((1, C1), lambda b, t: (0, 0)),
        ],
        out_specs=[
            pl.BlockSpec((1, TILE, C1), lambda b, t: (b, t, 0)),
            pl.BlockSpec((1, 2 * C1), lambda b, t: (0, 0)),
            pl.BlockSpec((2 * C1, 2 * C1), lambda b, t: (0, 0)),
        ],
        out_shape=[
            jax.ShapeDtypeStruct((B, N, C1), F32),
            jax.ShapeDtypeStruct((1, 2 * C1), F32),
            jax.ShapeDtypeStruct((2 * C1, 2 * C1), F32),
        ],
    )(gpk, z, s1, t1)

    s2, t2 = pl.pallas_call(
        lambda mu, mm, w2, g2r, b2r, s2o, t2o: _p2b_kernel(
            mu, mm, w2, g2r, b2r, m_edges, s2o, t2o),
        out_shape=[
            jax.ShapeDtypeStruct((1, C2), F32),
            jax.ShapeDtypeStruct((1, C2), F32),
        ],
    )(muhr, mhr, W2, g2.reshape(C2, 1), b2.reshape(C2, 1))

    w2p = jnp.zeros((2 * C1, 2 * C2), F32)
    w2p = w2p.at[:C1, :C2].set(W2.T)
    w2p = w2p.at[C1:, C2:].set(W2.T)

    x2, muv, mv = pl.pallas_call(
        _p3_kernel,
        grid=(B, T),
        in_specs=[
            pl.BlockSpec((1, K // 2, TILE, 2 * C1),
                         lambda b, t: (b, 0, t, 0)),
            pl.BlockSpec((1, TILE, C1), lambda b, t: (b, t, 0)),
            pl.BlockSpec((1, C1), lambda b, t: (0, 0)),
            pl.BlockSpec((1, C1), lambda b, t: (0, 0)),
            pl.BlockSpec((1, C2), lambda b, t: (0, 0)),
            pl.BlockSpec((1, C2), lambda b, t: (0, 0)),
            pl.BlockSpec((2 * C1, 2 * C2), lambda b, t: (0, 0)),
            pl.BlockSpec((1, TILE, C1), lambda b, t: (b, t, 0)),
        ],
        out_specs=[
            pl.BlockSpec((1, TILE, C2), lambda b, t: (b, t, 0)),
            pl.BlockSpec((1, CV), lambda b, t: (0, 0)),
            pl.BlockSpec((CV, CV), lambda b, t: (0, 0)),
        ],
        out_shape=[
            jax.ShapeDtypeStruct((B, N, C2), F32),
            jax.ShapeDtypeStruct((1, CV), F32),
            jax.ShapeDtypeStruct((CV, CV), F32),
        ],
    )(gpk, z, s1, t1, s2, t2, w2p, x1)

    so, to = pl.pallas_call(
        lambda mu, mm, wo, gor, bor, soo, too: _p3b_kernel(
            mu, mm, wo, gor, bor, B * N, soo, too),
        out_shape=[
            jax.ShapeDtypeStruct((CO, 1), F32),
            jax.ShapeDtypeStruct((CO, 1), F32),
        ],
    )(muv, mv, Wout, gout.reshape(CO, 1), bout.reshape(CO, 1))

    out = pl.pallas_call(
        _p4_kernel,
        grid=(B, T),
        in_specs=[
            pl.BlockSpec((1, TILE, C1), lambda b, t: (b, t, 0)),
            pl.BlockSpec((1, TILE, C2), lambda b, t: (b, t, 0)),
            pl.BlockSpec((CO, CV), lambda b, t: (0, 0)),
            pl.BlockSpec((CO, 1), lambda b, t: (0, 0)),
            pl.BlockSpec((CO, 1), lambda b, t: (0, 0)),
        ],
        out_specs=pl.BlockSpec((1, CO, TILE), lambda b, t: (b, 0, t)),
        out_shape=jax.ShapeDtypeStruct((B, CO, N), F32),
    )(x1, x2, Wout, so, to)
    return out


# speedup vs baseline: 2.4124x; 1.0660x over previous
import functools

import jax
import jax.numpy as jnp
from jax import lax
from jax.experimental import pallas as pl
from jax.experimental.pallas import tpu as pltpu
from jax.experimental.pallas import tpu_sc as plsc

K = 20
TILE = 512
F32 = jnp.float32


def _dot(a, b, dims):
    return lax.dot_general(a, b, (dims, ((), ())), preferred_element_type=F32)


def _p0_kernel(x_ref, w1a_ref, wz_ref, y_ref, z_ref):
    xb = x_ref[0]
    y_ref[0] = _dot(xb, w1a_ref[...], ((0,), (1,)))
    z_ref[0] = _dot(xb, wz_ref[...], ((0,), (1,)))


def _p1_kernel(xt_ref, xb_ref, yb_ref, z_ref, idx_ref, sumh_ref, sumsq_ref):
    b = pl.program_id(0)
    t = pl.program_id(1)
    n_total = xb_ref.shape[2]
    xt = xt_ref[0]
    xb = xb_ref[0]
    yb = yb_ref[0]
    z = z_ref[0]

    inner = _dot(xt, xb, ((0,), (0,)))
    xxp = jnp.sum(xt * xt, axis=0)
    xxb = jnp.sum(xb * xb, axis=0)
    d = 2.0 * inner - xxp[:, None] - xxb[None, :]

    iota = lax.broadcasted_iota(jnp.int32, d.shape, 1)
    neg = jnp.float32(-jnp.inf)
    for j in range(K):
        m = jnp.argmax(d, axis=1).astype(jnp.int32)[:, None]
        d = jnp.where(iota == m, neg, d)
        idx_ref[0, :, pl.ds(j, 1)] = m + b * n_total

    khot = (d == neg).astype(F32)
    s = _dot(khot, yb, ((1,), (0,)))
    s2 = _dot(khot, yb * yb, ((1,), (0,)))
    sh = jnp.sum(s, axis=0) + K * jnp.sum(z, axis=0)
    sq = (jnp.sum(s2, axis=0) + 2.0 * jnp.sum(s * z, axis=0)
          + K * jnp.sum(z * z, axis=0))

    @pl.when(jnp.logical_and(b == 0, t == 0))
    def _init():
        sumh_ref[...] = jnp.zeros_like(sumh_ref)
        sumsq_ref[...] = jnp.zeros_like(sumsq_ref)
    sumh_ref[0, :] = sumh_ref[0, :] + sh
    sumsq_ref[0, :] = sumsq_ref[0, :] + sq


def _sc_gather_kernel(e_per_w, chunk, y_hbm, idx_hbm, g_hbm,
                      idx_v, rows0, rows1, sem0, sem1):
    nc = 2
    wid = lax.axis_index("s") * nc + lax.axis_index("c")
    base = wid * e_per_w
    n_chunks = e_per_w // chunk
    rows = (rows0, rows1)
    sems = (sem0, sem1)
    pltpu.sync_copy(idx_hbm.at[pl.ds(base, e_per_w)], idx_v)
    cp0 = pltpu.async_copy(y_hbm.at[idx_v.at[pl.ds(0, chunk)]],
                           rows0, sem0)
    copies = [cp0]
    for i in range(n_chunks):
        if i + 1 < n_chunks:
            copies.append(pltpu.async_copy(
                y_hbm.at[idx_v.at[pl.ds((i + 1) * chunk, chunk)]],
                rows[(i + 1) % 2], sems[(i + 1) % 2]))
        copies[i].wait()
        pltpu.sync_copy(rows[i % 2], g_hbm.at[pl.ds(base + i * chunk, chunk)])


def _p2_kernel(g_ref, z_ref, s1_ref, t1_ref,
               x1_ref, muhr_ref, mhr_ref):
    b = pl.program_id(0)
    t = pl.program_id(1)
    c1 = z_ref.shape[2]
    z = z_ref[0]
    z2 = jnp.concatenate([z, z], axis=1)
    s1 = s1_ref[...]
    t1 = t1_ref[...]
    s12 = jnp.concatenate([s1, s1], axis=1)
    t12 = jnp.concatenate([t1, t1], axis=1)
    hps = []
    for j2 in range(K // 2):
        hp = jnp.maximum((g_ref[0, j2] + z2) * s12 + t12, 0.0)
        hps.append(hp)
    mp = hps[0]
    for j2 in range(1, K // 2):
        mp = jnp.maximum(mp, hps[j2])
    mx = jnp.maximum(mp[:, :c1], mp[:, c1:])
    mx2 = jnp.concatenate([mx, mx], axis=1)
    ssum = jnp.zeros_like(mp)
    num = jnp.zeros_like(mp)
    mu = jnp.zeros((2 * c1,), F32)
    mm = jnp.zeros((2 * c1, 2 * c1), F32)
    for j2 in range(K // 2):
        e = jnp.exp(hps[j2] - mx2)
        ssum = ssum + e
        num = num + hps[j2] * e
        mu = mu + jnp.sum(hps[j2], axis=0)
        mm = mm + _dot(hps[j2], hps[j2], ((0,), (0,)))
    x1_ref[0] = ((num[:, :c1] + num[:, c1:])
                 / (ssum[:, :c1] + ssum[:, c1:]))

    @pl.when(jnp.logical_and(b == 0, t == 0))
    def _init():
        muhr_ref[...] = jnp.zeros_like(muhr_ref)
        mhr_ref[...] = jnp.zeros_like(mhr_ref)
    muhr_ref[0, :] = muhr_ref[0, :] + mu
    mhr_ref[...] = mhr_ref[...] + mm


def _p2b_kernel(mu_ref, mm_ref, w2_ref, g2_ref, b2_ref, n_samples,
                s2_ref, t2_ref):
    minv = jnp.float32(1.0 / n_samples)
    c1 = w2_ref.shape[1]
    w2 = w2_ref[...]
    mup = mu_ref[...]
    mmp = mm_ref[...]
    mean_hr = (mup[:, :c1] + mup[:, c1:]) * minv
    mm_hr = (mmp[:c1, :c1] + mmp[c1:, c1:]) * minv
    mean_c = _dot(w2, mean_hr, ((1,), (1,)))
    u = _dot(w2, mm_hr, ((1,), (0,)))
    e2 = jnp.sum(u * w2, axis=1, keepdims=True)
    var = e2 - mean_c * mean_c
    rstd = lax.rsqrt(var + 1e-5)
    gg = g2_ref[...]
    sc = gg * rstd
    tc = b2_ref[...] - mean_c * sc
    s2_ref[...] = jnp.reshape(sc, s2_ref.shape)
    t2_ref[...] = jnp.reshape(tc, t2_ref.shape)


def _p3_kernel(g_ref, z_ref, s1_ref, t1_ref, s2_ref, t2_ref, w2p_ref, x1_ref,
               x2_ref, muv_ref, mv_ref):
    b = pl.program_id(0)
    t = pl.program_id(1)
    c1 = z_ref.shape[2]
    c2 = s2_ref.shape[1]
    z = z_ref[0]
    z2 = jnp.concatenate([z, z], axis=1)
    s1 = s1_ref[...]
    t1 = t1_ref[...]
    s12 = jnp.concatenate([s1, s1], axis=1)
    t12 = jnp.concatenate([t1, t1], axis=1)
    s2 = s2_ref[...]
    t2 = t2_ref[...]
    s22 = jnp.concatenate([s2, s2], axis=1)
    t22 = jnp.concatenate([t2, t2], axis=1)
    w2p = w2p_ref[...]
    hr2s = []
    for j2 in range(K // 2):
        hp = jnp.maximum((g_ref[0, j2] + z2) * s12 + t12, 0.0)
        h2p = _dot(hp, w2p, ((1,), (0,)))
        hr2s.append(jnp.maximum(h2p * s22 + t22, 0.0))
    mp = hr2s[0]
    for j2 in range(1, K // 2):
        mp = jnp.maximum(mp, hr2s[j2])
    mx = jnp.maximum(mp[:, :c2], mp[:, c2:])
    mx2 = jnp.concatenate([mx, mx], axis=1)
    ssum = jnp.zeros_like(mp)
    num = jnp.zeros_like(mp)
    for j2 in range(K // 2):
        e = jnp.exp(hr2s[j2] - mx2)
        ssum = ssum + e
        num = num + hr2s[j2] * e
    x2 = ((num[:, :c2] + num[:, c2:])
          / (ssum[:, :c2] + ssum[:, c2:]))
    x2_ref[0] = x2

    v = jnp.concatenate([x1_ref[0], x2], axis=1)
    mu = jnp.sum(v, axis=0)
    mm = _dot(v, v, ((0,), (0,)))

    @pl.when(jnp.logical_and(b == 0, t == 0))
    def _init():
        muv_ref[...] = jnp.zeros_like(muv_ref)
        mv_ref[...] = jnp.zeros_like(mv_ref)
    muv_ref[0, :] = muv_ref[0, :] + mu
    mv_ref[...] = mv_ref[...] + mm


def _p3b_kernel(mu_ref, mm_ref, wo_ref, go_ref, bo_ref, n_samples,
                so_ref, to_ref):
    minv = jnp.float32(1.0 / n_samples)
    wo = wo_ref[...]
    mean_v = mu_ref[...] * minv
    mean_c = _dot(wo, mean_v, ((1,), (1,)))
    u = _dot(wo, mm_ref[...] * minv, ((1,), (0,)))
    e2 = jnp.sum(u * wo, axis=1, keepdims=True)
    var = e2 - mean_c * mean_c
    rstd = lax.rsqrt(var + 1e-5)
    go = go_ref[...]
    so_ref[...] = go * rstd
    to_ref[...] = bo_ref[...] - mean_c * go * rstd


def _p4_kernel(x1_ref, x2_ref, wo_ref, so_ref, to_ref, out_ref):
    v = jnp.concatenate([x1_ref[0], x2_ref[0]], axis=1)
    o = _dot(wo_ref[...], v, ((1,), (1,)))
    out_ref[0] = jnp.maximum(o * so_ref[...] + to_ref[...], 0.0)


def kernel(x, W1, g1, b1, W2, g2, b2, Wout, gout, bout):
    B, C, N = x.shape
    C1 = W1.shape[0]
    C2 = W2.shape[0]
    CO = Wout.shape[0]
    CV = Wout.shape[1]
    T = N // TILE
    W1a = W1[:, :C]
    Wz = W1[:, C:] - W1[:, :C]

    y, z = pl.pallas_call(
        _p0_kernel,
        grid=(B,),
        in_specs=[
            pl.BlockSpec((1, C, N), lambda b: (b, 0, 0)),
            pl.BlockSpec((C1, C), lambda b: (0, 0)),
            pl.BlockSpec((C1, C), lambda b: (0, 0)),
        ],
        out_specs=[
            pl.BlockSpec((1, N, C1), lambda b: (b, 0, 0)),
            pl.BlockSpec((1, N, C1), lambda b: (b, 0, 0)),
        ],
        out_shape=[
            jax.ShapeDtypeStruct((B, N, C1), F32),
            jax.ShapeDtypeStruct((B, N, C1), F32),
        ],
    )(x, W1a, Wz)

    idx, sumh, sumsq = pl.pallas_call(
        _p1_kernel,
        grid=(B, T),
        in_specs=[
            pl.BlockSpec((1, C, TILE), lambda b, t: (b, 0, t)),
            pl.BlockSpec((1, C, N), lambda b, t: (b, 0, 0)),
            pl.BlockSpec((1, N, C1), lambda b, t: (b, 0, 0)),
            pl.BlockSpec((1, TILE, C1), lambda b, t: (b, t, 0)),
        ],
        out_specs=[
            pl.BlockSpec((1, TILE, K), lambda b, t: (b, t, 0)),
            pl.BlockSpec((1, C1), lambda b, t: (0, 0)),
            pl.BlockSpec((1, C1), lambda b, t: (0, 0)),
        ],
        out_shape=[
            jax.ShapeDtypeStruct((B, N, K), jnp.int32),
            jax.ShapeDtypeStruct((1, C1), F32),
            jax.ShapeDtypeStruct((1, C1), F32),
        ],
    )(x, x, y, z)

    n_edges = B * N * K
    n_workers = 32
    e_per_w = n_edges // n_workers
    chunk = 640
    mesh = plsc.VectorSubcoreMesh(core_axis_name="c", subcore_axis_name="s")
    gather = pl.kernel(
        functools.partial(_sc_gather_kernel, e_per_w, chunk),
        mesh=mesh,
        out_type=jax.ShapeDtypeStruct((n_edges, C1), F32),
        scratch_types=[
            pltpu.VMEM((e_per_w,), jnp.int32),
            pltpu.VMEM((chunk, C1), F32),
            pltpu.VMEM((chunk, C1), F32),
            pltpu.SemaphoreType.DMA,
            pltpu.SemaphoreType.DMA,
        ],
        compiler_params=pltpu.CompilerParams(use_tc_tiling_on_sc=False),
    )
    idxp = jnp.transpose(idx.reshape(B, N, K // 2, 2),
                         (0, 2, 1, 3)).reshape(n_edges)
    g = gather(y.reshape(B * N, C1), idxp)
    gpk = g.reshape(B, K // 2, N, 2 * C1)

    m_edges = B * N * K
    mean1 = sumh / m_edges
    var1 = sumsq / m_edges - mean1 * mean1
    rstd1 = 1.0 / jnp.sqrt(var1 + 1e-5)
    s1 = g1.reshape(1, C1) * rstd1
    t1 = b1.reshape(1, C1) - mean1 * s1

    x1, muhr, mhr = pl.pallas_call(
        _p2_kernel,
        grid=(B, T),
        in_specs=[
            pl.BlockSpec((1, K // 2, TILE, 2 * C1),
                         lambda b, t: (b, 0, t, 0)),
            pl.BlockSpec((1, TILE, C1), lambda b, t: (b, t, 0)),
            pl.BlockSpec((1, C1), lambda b, t: (0, 0)),
            pl.BlockSpec((1, C1), lambda b, t: (0, 0)),
        ],
        out_specs=[
            pl.BlockSpec((1, TILE, C1), lambda b, t: (b, t, 0)),
            pl.BlockSpec((1, 2 * C1), lambda b, t: (0, 0)),
            pl.BlockSpec((2 * C1, 2 * C1), lambda b, t: (0, 0)),
        ],
        out_shape=[
            jax.ShapeDtypeStruct((B, N, C1), F32),
            jax.ShapeDtypeStruct((1, 2 * C1), F32),
            jax.ShapeDtypeStruct((2 * C1, 2 * C1), F32),
        ],
    )(gpk, z, s1, t1)

    s2, t2 = pl.pallas_call(
        lambda mu, mm, w2, g2r, b2r, s2o, t2o: _p2b_kernel(
            mu, mm, w2, g2r, b2r, m_edges, s2o, t2o),
        out_shape=[
            jax.ShapeDtypeStruct((1, C2), F32),
            jax.ShapeDtypeStruct((1, C2), F32),
        ],
    )(muhr, mhr, W2, g2.reshape(C2, 1), b2.reshape(C2, 1))

    w2p = jnp.zeros((2 * C1, 2 * C2), F32)
    w2p = w2p.at[:C1, :C2].set(W2.T)
    w2p = w2p.at[C1:, C2:].set(W2.T)

    x2, muv, mv = pl.pallas_call(
        _p3_kernel,
        grid=(B, T),
        in_specs=[
            pl.BlockSpec((1, K // 2, TILE, 2 * C1),
                         lambda b, t: (b, 0, t, 0)),
            pl.BlockSpec((1, TILE, C1), lambda b, t: (b, t, 0)),
            pl.BlockSpec((1, C1), lambda b, t: (0, 0)),
            pl.BlockSpec((1, C1), lambda b, t: (0, 0)),
            pl.BlockSpec((1, C2), lambda b, t: (0, 0)),
            pl.BlockSpec((1, C2), lambda b, t: (0, 0)),
            pl.BlockSpec((2 * C1, 2 * C2), lambda b, t: (0, 0)),
            pl.BlockSpec((1, TILE, C1), lambda b, t: (b, t, 0)),
        ],
        out_specs=[
            pl.BlockSpec((1, TILE, C2), lambda b, t: (b, t, 0)),
            pl.BlockSpec((1, CV), lambda b, t: (0, 0)),
            pl.BlockSpec((CV, CV), lambda b, t: (0, 0)),
        ],
        out_shape=[
            jax.ShapeDtypeStruct((B, N, C2), F32),
            jax.ShapeDtypeStruct((1, CV), F32),
            jax.ShapeDtypeStruct((CV, CV), F32),
        ],
    )(gpk, z, s1, t1, s2, t2, w2p, x1)

    so, to = pl.pallas_call(
        lambda mu, mm, wo, gor, bor, soo, too: _p3b_kernel(
            mu, mm, wo, gor, bor, B * N, soo, too),
        out_shape=[
            jax.ShapeDtypeStruct((CO, 1), F32),
            jax.ShapeDtypeStruct((CO, 1), F32),
        ],
    )(muv, mv, Wout, gout.reshape(CO, 1), bout.reshape(CO, 1))

    out = pl.pallas_call(
        _p4_kernel,
        grid=(B, T),
        in_specs=[
            pl.BlockSpec((1, TILE, C1), lambda b, t: (b, t, 0)),
            pl.BlockSpec((1, TILE, C2), lambda b, t: (b, t, 0)),
            pl.BlockSpec((CO, CV), lambda b, t: (0, 0)),
            pl.BlockSpec((CO, 1), lambda b, t: (0, 0)),
            pl.BlockSpec((CO, 1), lambda b, t: (0, 0)),
        ],
        out_specs=pl.BlockSpec((1, CO, TILE), lambda b, t: (b, 0, t)),
        out_shape=jax.ShapeDtypeStruct((B, CO, N), F32),
    )(x1, x2, Wout, so, to)
    return out
